# baseline (speedup 1.0000x reference)
import jax
import jax.numpy as jnp
from jax import lax
from jax.experimental import pallas as pl
from jax.experimental.pallas import tpu as pltpu
from jax.experimental.pallas import tpu_sc as plsc

B, L, V, D, C = 4096, 50, 1000000, 64, 10
N = B * L
NW = 32
PER_W = N // NW
CHUNK = 128
NCHUNK = PER_W // CHUNK
NB = 10
NGROUP = NCHUNK // NB


def _sc_gather_body(word_hbm, table_hbm, out_hbm, idx_v, rows_v, gsem, wsem):
    cid = lax.axis_index("c")
    sid = lax.axis_index("s")
    wid = sid * 2 + cid
    pltpu.sync_copy(word_hbm.at[wid], idx_v)

    def group(g, carry):
        base = g * NB
        gets = [
            pltpu.async_copy(table_hbm.at[idx_v.at[base + t]], rows_v.at[t], gsem)
            for t in range(NB)
        ]
        for cp in gets:
            cp.wait()
        puts = [
            pltpu.async_copy(rows_v.at[t], out_hbm.at[wid * NCHUNK + base + t], wsem)
            for t in range(NB)
        ]
        for cp in puts:
            cp.wait()
        return carry

    lax.fori_loop(0, NGROUP, group, 0)


def _sc_gather(widx, table):
    mesh = plsc.VectorSubcoreMesh(core_axis_name="c", subcore_axis_name="s")
    kern = pl.kernel(
        _sc_gather_body,
        mesh=mesh,
        out_type=jax.ShapeDtypeStruct((NW * NCHUNK, CHUNK, D), jnp.float32),
        scratch_types=[
            pltpu.VMEM((NCHUNK, CHUNK), jnp.int32),
            pltpu.VMEM((NB, CHUNK, D), jnp.float32),
            pltpu.SemaphoreType.DMA,
            pltpu.SemaphoreType.DMA,
        ],
        compiler_params=pltpu.CompilerParams(use_tc_tiling_on_sc=False),
    )
    return kern(widx, table)


def _tc_combine_body(x_ref, wv_ref, b_ref, out_ref):
    bb = x_ref.shape[0]
    x = x_ref[...].reshape(bb * L, D)
    z = lax.dot_general(
        x, wv_ref[...], (((1,), (0,)), ((), ())),
        preferred_element_type=jnp.float32,
    ).reshape(bb, L, 5 * C)
    zp = jnp.pad(z, ((0, 0), (2, 2), (0, 0)))
    acc = jnp.broadcast_to(b_ref[...].reshape(1, 1, C), (bb, L, C))
    for i in range(5):
        acc = acc + zp[:, i:i + L, C * i:C * (i + 1)]
    out_ref[...] = acc


def _tc_combine(x, wv, b2):
    bb = 256
    return pl.pallas_call(
        _tc_combine_body,
        grid=(B // bb,),
        in_specs=[
            pl.BlockSpec((bb, L, D), lambda i: (i, 0, 0)),
            pl.BlockSpec((D, 5 * C), lambda i: (0, 0)),
            pl.BlockSpec((1, C), lambda i: (0, 0)),
        ],
        out_specs=pl.BlockSpec((bb, L, C), lambda i: (i, 0, 0)),
        out_shape=jax.ShapeDtypeStruct((B, L, C), jnp.float32),
    )(x, wv, b2)


def kernel(word, table, W, b):
    widx = word.astype(jnp.int32).reshape(NW, NCHUNK, CHUNK)
    x = _sc_gather(widx, table).reshape(B, L, D)
    wv = W.reshape(5, D, C).transpose(1, 0, 2).reshape(D, 5 * C)
    return _tc_combine(x, wv, b.reshape(1, C))


# speedup vs baseline: 1.0202x; 1.0202x over previous
import jax
import jax.numpy as jnp
from jax import lax
from jax.experimental import pallas as pl
from jax.experimental.pallas import tpu as pltpu
from jax.experimental.pallas import tpu_sc as plsc

B, L, V, D, C = 4096, 50, 1000000, 64, 10
LP = 56
PAIR = 2 * D
NW = 32
SENT_W = B // NW
NB = 4
NGROUP = SENT_W // NB


def _sc_gather_body(word_hbm, table_hbm, out_hbm, idx_v, rows_v, gsem, wsem):
    cid = lax.axis_index("c")
    sid = lax.axis_index("s")
    wid = sid * 2 + cid
    base_s = wid * SENT_W
    pltpu.sync_copy(word_hbm.at[wid], idx_v)

    def do_group(g, buf):
        s0 = g * NB
        gets = [
            pltpu.async_copy(
                table_hbm.at[idx_v.at[s0 + t]],
                rows_v.at[buf, t, pl.ds(0, L)],
                gsem,
            )
            for t in range(NB)
        ]
        for cp in gets:
            cp.wait()
        return pltpu.async_copy(
            rows_v.at[buf], out_hbm.at[pl.ds(base_s + s0, NB)], wsem
        )

    put0 = do_group(0, 0)
    put1 = do_group(1, 1)

    def pair(h, carry):
        g = 2 * h
        put0.wait()
        p0 = do_group(g + 2, 0)
        put1.wait()
        p1 = do_group(g + 3, 1)
        return carry

    lax.fori_loop(0, (NGROUP - 2) // 2, pair, 0)
    put0.wait()
    put1.wait()


def _sc_gather(widx, table2):
    mesh = plsc.VectorSubcoreMesh(core_axis_name="c", subcore_axis_name="s")
    kern = pl.kernel(
        _sc_gather_body,
        mesh=mesh,
        out_type=jax.ShapeDtypeStruct((B, LP, PAIR), jnp.float32),
        scratch_types=[
            pltpu.VMEM((SENT_W, L), jnp.int32),
            pltpu.VMEM((2, NB, LP, PAIR), jnp.float32),
            pltpu.SemaphoreType.DMA,
            pltpu.SemaphoreType.DMA,
        ],
    )
    return kern(widx, table2)


def _tc_combine_body(x2_ref, w_ref, wv_ref, b_ref, out_ref):
    bb = w_ref.shape[0]
    x3 = x2_ref[...]
    qf = (w_ref[...] & 1).astype(jnp.float32)
    qf = jnp.pad(qf, ((0, 0), (0, LP - L)))[:, :, None]
    x = x3[:, :, :D] * (1.0 - qf) + x3[:, :, D:] * qf
    z = lax.dot_general(
        x.reshape(bb * LP, D), wv_ref[...], (((1,), (0,)), ((), ())),
        preferred_element_type=jnp.float32,
    ).reshape(bb, LP, 5 * C)[:, :L, :]
    zp = jnp.pad(z, ((0, 0), (2, 2), (0, 0)))
    acc = jnp.broadcast_to(b_ref[...].reshape(1, 1, C), (bb, L, C))
    for i in range(5):
        acc = acc + zp[:, i:i + L, C * i:C * (i + 1)]
    out_ref[...] = acc


def _tc_combine(x2, word, wv, b2):
    bb = 256
    return pl.pallas_call(
        _tc_combine_body,
        grid=(B // bb,),
        in_specs=[
            pl.BlockSpec((bb, LP, PAIR), lambda i: (i, 0, 0)),
            pl.BlockSpec((bb, L), lambda i: (i, 0)),
            pl.BlockSpec((D, 5 * C), lambda i: (0, 0)),
            pl.BlockSpec((1, C), lambda i: (0, 0)),
        ],
        out_specs=pl.BlockSpec((bb, L, C), lambda i: (i, 0, 0)),
        out_shape=jax.ShapeDtypeStruct((B, L, C), jnp.float32),
    )(x2, word, wv, b2)


def kernel(word, table, W, b):
    word = word.astype(jnp.int32)
    widx = (word >> 1).reshape(NW, SENT_W, L)
    table2 = table.reshape(V // 2, PAIR)
    x2 = _sc_gather(widx, table2)
    wv = W.reshape(5, D, C).transpose(1, 0, 2).reshape(D, 5 * C)
    return _tc_combine(x2, word, wv, b.reshape(1, C))
